# baseline (speedup 1.0000x reference)
import functools
import math

import jax
import jax.numpy as jnp
from jax import lax
from jax.experimental import pallas as pl
from jax.experimental.pallas import tpu as pltpu
from jax.experimental.pallas import tpu_sc as plsc

N = 5000
R = 160
G = R // 16
W = 128
HALF_PI = float(math.pi / 2)
PI = float(math.pi)


def _atan_poly(a):
    s = a * a
    p = jnp.float32(-0.0117212)
    p = p * s + jnp.float32(0.05265332)
    p = p * s + jnp.float32(-0.11643287)
    p = p * s + jnp.float32(0.19354346)
    p = p * s + jnp.float32(-0.33262348)
    p = p * s + jnp.float32(0.99997726)
    return a * p


def _atan2(y, x):
    ax = jnp.abs(x)
    ay = jnp.abs(y)
    mx = jnp.maximum(ax, ay)
    mn = jnp.minimum(ax, ay)
    a = mn / jnp.where(mx == 0, jnp.float32(1.0), mx)
    r = _atan_poly(a)
    r = jnp.where(ay > ax, jnp.float32(HALF_PI) - r, r)
    r = jnp.where(x < 0, jnp.float32(PI) - r, r)
    return jnp.where(y < 0, -r, r)


def _atan(t):
    at = jnp.abs(t)
    inv = at > 1
    a = jnp.where(inv, jnp.float32(1.0) / jnp.where(at == 0, jnp.float32(1.0), at), at)
    r = _atan_poly(a)
    r = jnp.where(inv, jnp.float32(HALF_PI) - r, r)
    return jnp.where(t < 0, -r, r)


@functools.partial(
    pl.kernel,
    mesh=plsc.VectorSubcoreMesh(core_axis_name="c", subcore_axis_name="s"),
    compiler_params=pltpu.CompilerParams(needs_layout_passes=False,
                                         use_tc_tiling_on_sc=False,
                                         skip_device_barrier=True),
    out_type=jax.ShapeDtypeStruct((N, 8), jnp.float32),
    scratch_types=[
        pltpu.VMEM((R * W,), jnp.float32),
        pltpu.VMEM((R,), jnp.int32),
        pltpu.VMEM((R, 8), jnp.float32),
        pltpu.SemaphoreType.DMA,
    ],
)
def _sc_decode(in_hbm, lab_hbm, out_hbm, in_v, lab_v, out_v, sem):
    wid = lax.axis_index("s") * 2 + lax.axis_index("c")
    base = jnp.minimum(wid * R, N - R)

    cp0 = pltpu.async_copy(in_hbm.at[pl.ds(base * W, R * W)], in_v, sem)
    cp1 = pltpu.async_copy(lab_hbm.at[pl.ds(base, R)], lab_v, sem)
    cp0.wait()
    cp1.wait()

    iota = jnp.arange(16, dtype=jnp.int32)
    zero = jnp.zeros((16,), jnp.int32)

    def group(g, _):
        r = iota + g * 16
        rw = r * W
        lab = lab_v[pl.ds(g * 16, 16)]
        li = jnp.clip(lab - 1, 0, 2)

        c3 = rw + li * 3
        d0 = plsc.load_gather(in_v, [c3])
        d1 = plsc.load_gather(in_v, [c3 + 1])
        d2 = plsc.load_gather(in_v, [c3 + 2])
        is0 = li == 0
        is1 = li == 1
        m0 = jnp.where(is0, jnp.float32(3.88),
                       jnp.where(is1, jnp.float32(0.84), jnp.float32(1.76)))
        m1 = jnp.where(is0, jnp.float32(1.63),
                       jnp.where(is1, jnp.float32(0.60), jnp.float32(0.60)))
        m2 = jnp.where(is0, jnp.float32(1.53),
                       jnp.where(is1, jnp.float32(1.76), jnp.float32(1.73)))
        pd0 = jnp.exp(d0 / jnp.float32(5.0)) * m0
        pd1 = jnp.exp(d1 / jnp.float32(5.0)) * m1
        pd2 = jnp.exp(d2 / jnp.float32(5.0)) * m2

        l0 = plsc.load_gather(in_v, [rw + 9])
        l1 = plsc.load_gather(in_v, [rw + 10])
        bin1 = l1 > l0
        bcol = rw + jnp.where(bin1, jnp.int32(13), jnp.int32(11))
        sn = plsc.load_gather(in_v, [bcol])
        cs = plsc.load_gather(in_v, [bcol + 1])
        alpha = _atan2(sn, cs) + jnp.where(bin1, jnp.float32(HALF_PI),
                                           jnp.float32(-HALF_PI))

        c2 = rw + li * 2 + 15
        x = plsc.load_gather(in_v, [c2]) / jnp.float32(10.0)
        y = plsc.load_gather(in_v, [c2 + 1]) / jnp.float32(10.0) + jnp.float32(30.0)
        z = plsc.load_gather(in_v, [c2 + 2]) / jnp.float32(10.0)

        ry = alpha + _atan(x / y)

        for j, v in enumerate((ry, pd0, pd1, pd2, x, y, z, alpha)):
            plsc.store_scatter(out_v, [r, zero + j], v)
        return 0

    lax.fori_loop(0, G, group, 0, unroll=2)

    pltpu.sync_copy(out_v, out_hbm.at[pl.ds(base, R)])


def kernel(box3d_dim_regression, box3d_rotation_logits, box3d_rotation_regression,
           box3d_localization_center, labels):
    packed = jnp.concatenate(
        [box3d_dim_regression, box3d_rotation_logits, box3d_rotation_regression,
         box3d_localization_center,
         jnp.zeros((N, W - 24), jnp.float32)], axis=1)
    return _sc_decode(packed.reshape(-1), labels.astype(jnp.int32))


# speedup vs baseline: 1.5473x; 1.5473x over previous
import functools
import math

import jax
import jax.numpy as jnp
from jax import lax
from jax.experimental import pallas as pl
from jax.experimental.pallas import tpu as pltpu
from jax.experimental.pallas import tpu_sc as plsc

N = 5000
R = 160
G = R // 16
W = 24
HALF_PI = float(math.pi / 2)
PI = float(math.pi)


def _atan_poly(a):
    s = a * a
    p = jnp.float32(-0.0117212)
    p = p * s + jnp.float32(0.05265332)
    p = p * s + jnp.float32(-0.11643287)
    p = p * s + jnp.float32(0.19354346)
    p = p * s + jnp.float32(-0.33262348)
    p = p * s + jnp.float32(0.99997726)
    return a * p


def _atan2(y, x):
    ax = jnp.abs(x)
    ay = jnp.abs(y)
    mx = jnp.maximum(ax, ay)
    mn = jnp.minimum(ax, ay)
    a = mn / jnp.where(mx == 0, jnp.float32(1.0), mx)
    r = _atan_poly(a)
    r = jnp.where(ay > ax, jnp.float32(HALF_PI) - r, r)
    r = jnp.where(x < 0, jnp.float32(PI) - r, r)
    return jnp.where(y < 0, -r, r)


def _atan(t):
    at = jnp.abs(t)
    inv = at > 1
    a = jnp.where(inv, jnp.float32(1.0) / jnp.where(at == 0, jnp.float32(1.0), at), at)
    r = _atan_poly(a)
    r = jnp.where(inv, jnp.float32(HALF_PI) - r, r)
    return jnp.where(t < 0, -r, r)


@functools.partial(
    pl.kernel,
    mesh=plsc.VectorSubcoreMesh(core_axis_name="c", subcore_axis_name="s"),
    compiler_params=pltpu.CompilerParams(needs_layout_passes=False,
                                         use_tc_tiling_on_sc=False,
                                         skip_device_barrier=True),
    out_type=jax.ShapeDtypeStruct((N, 8), jnp.float32),
    scratch_types=[
        pltpu.VMEM((R, W), jnp.float32),
        pltpu.VMEM((R,), jnp.int32),
        pltpu.VMEM((R, 8), jnp.float32),
        pltpu.SemaphoreType.DMA,
    ],
)
def _sc_decode(in_hbm, lab_hbm, out_hbm, in_v, lab_v, out_v, sem):
    wid = lax.axis_index("s") * 2 + lax.axis_index("c")
    base = jnp.minimum(wid * R, N - R)

    cp0 = pltpu.async_copy(in_hbm.at[pl.ds(base, R)], in_v, sem)
    cp1 = pltpu.async_copy(lab_hbm.at[pl.ds(base, R)], lab_v, sem)
    cp0.wait()
    cp1.wait()

    iota = jnp.arange(16, dtype=jnp.int32)
    zero = jnp.zeros((16,), jnp.int32)

    def group(g, _):
        r = iota + g * 16
        lab = lab_v[pl.ds(g * 16, 16)]
        li = jnp.clip(lab - 1, 0, 2)

        c3 = li * 3
        d0 = plsc.load_gather(in_v, [r, c3])
        d1 = plsc.load_gather(in_v, [r, c3 + 1])
        d2 = plsc.load_gather(in_v, [r, c3 + 2])
        is0 = li == 0
        is1 = li == 1
        m0 = jnp.where(is0, jnp.float32(3.88),
                       jnp.where(is1, jnp.float32(0.84), jnp.float32(1.76)))
        m1 = jnp.where(is0, jnp.float32(1.63),
                       jnp.where(is1, jnp.float32(0.60), jnp.float32(0.60)))
        m2 = jnp.where(is0, jnp.float32(1.53),
                       jnp.where(is1, jnp.float32(1.76), jnp.float32(1.73)))
        pd0 = jnp.exp(d0 / jnp.float32(5.0)) * m0
        pd1 = jnp.exp(d1 / jnp.float32(5.0)) * m1
        pd2 = jnp.exp(d2 / jnp.float32(5.0)) * m2

        l0 = plsc.load_gather(in_v, [r, zero + 9])
        l1 = plsc.load_gather(in_v, [r, zero + 10])
        bin1 = l1 > l0
        bcol = jnp.where(bin1, jnp.int32(13), jnp.int32(11))
        sn = plsc.load_gather(in_v, [r, bcol])
        cs = plsc.load_gather(in_v, [r, bcol + 1])
        alpha = _atan2(sn, cs) + jnp.where(bin1, jnp.float32(HALF_PI),
                                           jnp.float32(-HALF_PI))

        c2 = li * 2 + 15
        x = plsc.load_gather(in_v, [r, c2]) / jnp.float32(10.0)
        y = plsc.load_gather(in_v, [r, c2 + 1]) / jnp.float32(10.0) + jnp.float32(30.0)
        z = plsc.load_gather(in_v, [r, c2 + 2]) / jnp.float32(10.0)

        ry = alpha + _atan(x / y)

        for j, v in enumerate((ry, pd0, pd1, pd2, x, y, z, alpha)):
            plsc.store_scatter(out_v, [r, zero + j], v)
        return 0

    lax.fori_loop(0, G, group, 0, unroll=2)

    pltpu.sync_copy(out_v, out_hbm.at[pl.ds(base, R)])


def kernel(box3d_dim_regression, box3d_rotation_logits, box3d_rotation_regression,
           box3d_localization_center, labels):
    packed = jnp.concatenate(
        [box3d_dim_regression, box3d_rotation_logits, box3d_rotation_regression,
         box3d_localization_center], axis=1)
    return _sc_decode(packed, labels.astype(jnp.int32))


# speedup vs baseline: 1.5509x; 1.0023x over previous
import functools
import math

import jax
import jax.numpy as jnp
from jax import lax
from jax.experimental import pallas as pl
from jax.experimental.pallas import tpu as pltpu
from jax.experimental.pallas import tpu_sc as plsc

N = 5000
R = 160
G = R // 16
W = 24
HALF_PI = float(math.pi / 2)
PI = float(math.pi)


def _atan_poly(a):
    s = a * a
    p = jnp.float32(-0.0117212)
    p = p * s + jnp.float32(0.05265332)
    p = p * s + jnp.float32(-0.11643287)
    p = p * s + jnp.float32(0.19354346)
    p = p * s + jnp.float32(-0.33262348)
    p = p * s + jnp.float32(0.99997726)
    return a * p


def _atan2(y, x):
    ax = jnp.abs(x)
    ay = jnp.abs(y)
    mx = jnp.maximum(ax, ay)
    mn = jnp.minimum(ax, ay)
    a = mn / jnp.where(mx == 0, jnp.float32(1.0), mx)
    r = _atan_poly(a)
    r = jnp.where(ay > ax, jnp.float32(HALF_PI) - r, r)
    r = jnp.where(x < 0, jnp.float32(PI) - r, r)
    return jnp.where(y < 0, -r, r)


def _atan(t):
    at = jnp.abs(t)
    inv = at > 1
    a = jnp.where(inv, jnp.float32(1.0) / jnp.where(at == 0, jnp.float32(1.0), at), at)
    r = _atan_poly(a)
    r = jnp.where(inv, jnp.float32(HALF_PI) - r, r)
    return jnp.where(t < 0, -r, r)


@functools.partial(
    pl.kernel,
    mesh=plsc.VectorSubcoreMesh(core_axis_name="c", subcore_axis_name="s"),
    compiler_params=pltpu.CompilerParams(needs_layout_passes=False,
                                         use_tc_tiling_on_sc=False,
                                         skip_device_barrier=True),
    out_type=jax.ShapeDtypeStruct((N, 8), jnp.float32),
    scratch_types=[
        pltpu.VMEM((R * W,), jnp.float32),
        pltpu.VMEM((R,), jnp.int32),
        pltpu.VMEM((R, 8), jnp.float32),
        pltpu.SemaphoreType.DMA,
    ],
)
def _sc_decode(in_hbm, lab_hbm, out_hbm, in_v, lab_v, out_v, sem):
    wid = lax.axis_index("s") * 2 + lax.axis_index("c")
    base = jnp.minimum(wid * R, N - R)

    cp0 = pltpu.async_copy(in_hbm.at[pl.ds(base * W, R * W)], in_v, sem)
    cp1 = pltpu.async_copy(lab_hbm.at[pl.ds(base, R)], lab_v, sem)
    cp0.wait()
    cp1.wait()

    iota = jnp.arange(16, dtype=jnp.int32)
    zero = jnp.zeros((16,), jnp.int32)

    def group(g, _):
        r = iota + g * 16
        rw = r * W
        lab = lab_v[pl.ds(g * 16, 16)]
        li = jnp.clip(lab - 1, 0, 2)

        c3 = rw + li * 3
        d0 = plsc.load_gather(in_v, [c3])
        d1 = plsc.load_gather(in_v, [c3 + 1])
        d2 = plsc.load_gather(in_v, [c3 + 2])
        is0 = li == 0
        is1 = li == 1
        m0 = jnp.where(is0, jnp.float32(3.88),
                       jnp.where(is1, jnp.float32(0.84), jnp.float32(1.76)))
        m1 = jnp.where(is0, jnp.float32(1.63),
                       jnp.where(is1, jnp.float32(0.60), jnp.float32(0.60)))
        m2 = jnp.where(is0, jnp.float32(1.53),
                       jnp.where(is1, jnp.float32(1.76), jnp.float32(1.73)))
        pd0 = jnp.exp(d0 / jnp.float32(5.0)) * m0
        pd1 = jnp.exp(d1 / jnp.float32(5.0)) * m1
        pd2 = jnp.exp(d2 / jnp.float32(5.0)) * m2

        l0 = plsc.load_gather(in_v, [rw + 9])
        l1 = plsc.load_gather(in_v, [rw + 10])
        bin1 = l1 > l0
        bcol = rw + jnp.where(bin1, jnp.int32(13), jnp.int32(11))
        sn = plsc.load_gather(in_v, [bcol])
        cs = plsc.load_gather(in_v, [bcol + 1])
        alpha = _atan2(sn, cs) + jnp.where(bin1, jnp.float32(HALF_PI),
                                           jnp.float32(-HALF_PI))

        c2 = rw + li * 2 + 15
        x = plsc.load_gather(in_v, [c2]) / jnp.float32(10.0)
        y = plsc.load_gather(in_v, [c2 + 1]) / jnp.float32(10.0) + jnp.float32(30.0)
        z = plsc.load_gather(in_v, [c2 + 2]) / jnp.float32(10.0)

        ry = alpha + _atan(x / y)

        for j, v in enumerate((ry, pd0, pd1, pd2, x, y, z, alpha)):
            plsc.store_scatter(out_v, [r, zero + j], v)
        return 0

    lax.fori_loop(0, G, group, 0, unroll=2)

    pltpu.sync_copy(out_v, out_hbm.at[pl.ds(base, R)])


def kernel(box3d_dim_regression, box3d_rotation_logits, box3d_rotation_regression,
           box3d_localization_center, labels):
    packed = jnp.concatenate(
        [box3d_dim_regression, box3d_rotation_logits, box3d_rotation_regression,
         box3d_localization_center], axis=1)
    return _sc_decode(packed.reshape(-1), labels.astype(jnp.int32))
